# baseline (speedup 1.0000x reference)
import jax
import jax.numpy as jnp
from jax.experimental import pallas as pl
from jax.experimental.pallas import tpu as pltpu

_EMBED = 16
_H = 96
_N = 64
_E = 1088
_STEPS = 4
_NEG = -1e9
_BB = 16
_INTERPRET = False


def _relu(v):
    return jnp.maximum(v, 0.0)


_DN = (((1,), (0,)), ((), ()))


def _mx(a, b):
    return jax.lax.dot_general(a, b, _DN, preferred_element_type=jnp.float32)


def _iota(shape, dim):
    return jax.lax.broadcasted_iota(jnp.int32, shape, dim)


def _rrn_kernel(inp_ref, y_ref, edges_ref, edgest_ref, rc_ref,
                exw1i_ref, exw1rc_ref, w6_ref, w2a_ref, wlstm_ref,
                bias6_ref, bg_ref, lb1_ref, outw_ref, outb_ref,
                o_ref, loss_ref, scr_ref, fin_ref):
    R = _BB * _N
    f32 = jnp.float32

    eo = (inp_ref[...] == _iota((R, _EMBED), 1)).astype(f32)
    y1h = (y_ref[...] == _iota((R, _EMBED), 1)).astype(f32)
    erc = jnp.concatenate(
        [(rc_ref[:, 0:1] == _iota((_N, _EMBED), 1)).astype(f32),
         (rc_ref[:, 1:2] == _iota((_N, _EMBED), 1)).astype(f32)], axis=1)
    gcat = jnp.concatenate(
        [(edges_ref[:, 0:1] == _iota((_E, _N), 1)).astype(f32),
         (edges_ref[:, 1:2] == _iota((_E, _N), 1)).astype(f32)], axis=1)
    gdt = (edgest_ref[1:2, :] == _iota((_N, _E), 0)).astype(f32)

    exw1i = exw1i_ref[...]
    exw1rc = exw1rc_ref[...]
    exw2 = w6_ref[0]
    exw3 = w6_ref[1]
    mm2w = w6_ref[2]
    mm3w = w6_ref[3]
    li2w = w6_ref[4]
    li3w = w6_ref[5]
    mm1a = w2a_ref[0, 0:_H]
    mm1b = w2a_ref[0, _H:2 * _H]
    li1f = w2a_ref[1, 0:_H]
    li1x = w2a_ref[1, _H:2 * _H]
    wih = wlstm_ref[0]
    whh = wlstm_ref[1]
    exb2 = bias6_ref[0:1]
    exb3 = bias6_ref[1:2]
    mmb1 = bias6_ref[2:3]
    mmb2 = bias6_ref[3:4]
    lib2 = bias6_ref[4:5]
    lib3 = bias6_ref[5:6]
    bg = bg_ref[...]
    outw = outw_ref[...]
    outb = outb_ref[...]

    xrc1 = _mx(erc, exw1rc)
    x = _relu(_mx(eo, exw1i) + jnp.concatenate([xrc1] * _BB, axis=0))
    x = _relu(_mx(x, exw2) + exb2)
    x = _mx(x, exw3) + exb3
    lb1 = jnp.concatenate([lb1_ref[...]] * _BB, axis=0)
    xli = _mx(x, li1x) + lb1

    hm = x
    h = None
    c = None
    acc = jnp.zeros((), f32)
    B = range(_BB)
    for t in range(_STEPS):
        av = _mx(hm, mm1a) + mmb1
        bv = _mx(hm, mm1b)
        for j in B:
            abj = jnp.concatenate(
                [av[j * _N:(j + 1) * _N], bv[j * _N:(j + 1) * _N]], axis=0)
            scr_ref[pl.ds(j * _E, _E), :] = _mx(gcat, abj)
        m1 = _relu(scr_ref[...])
        m2 = _relu(_mx(m1, mm2w) + mmb2)
        msgs = _mx(m2, mm3w)
        for j in B:
            fin_ref[pl.ds(j * _N, _N), :] = _mx(
                gdt, msgs[j * _E:(j + 1) * _E])
        fin = fin_ref[...]
        li1 = _relu(_mx(fin, li1f) + xli)
        li2 = _relu(_mx(li1, li2w) + lib2)
        il = _mx(li2, li3w) + lib3
        gates = _mx(il, wih) + bg
        if t > 0:
            gates = gates + _mx(h, whh)
        ig = gates[:, 0:_H]
        fg = gates[:, _H:2 * _H]
        gg = gates[:, 2 * _H:3 * _H]
        og = gates[:, 3 * _H:4 * _H]
        newc = jax.nn.sigmoid(ig) * jnp.tanh(gg)
        if t > 0:
            newc = newc + jax.nn.sigmoid(fg) * c
        c = newc
        h = jax.nn.sigmoid(og) * jnp.tanh(newc)
        hm = h
        o16 = _mx(c, outw) + outb
        mmax = jnp.max(o16, axis=1, keepdims=True)
        lse = jnp.log(jnp.sum(jnp.exp(o16 - mmax), axis=1,
                              keepdims=True)) + mmax
        acc = acc + jnp.sum((o16 - lse) * y1h)

    o_ref[...] = o16.reshape(_BB, _N, _EMBED)
    loss_ref[...] = jnp.broadcast_to(acc, (1, 1, 128)).astype(f32)


def kernel(inp, y_true, edges, row_col,
           ex_w1, ex_b1, ex_w2, ex_b2, ex_w3, ex_b3,
           mm_w1, mm_b1, mm_w2, mm_b2, mm_w3, mm_b3,
           li_w1, li_b1, li_w2, li_b2, li_w3, li_b3,
           out_w, out_b, lstm_wih, lstm_whh, lstm_bih, lstm_bhh):
    f32 = jnp.float32
    bs = inp.shape[0]
    e = edges.shape[0]
    inp2 = inp.astype(jnp.int32).reshape(bs * _N, 1)
    y2 = y_true.astype(jnp.int32).reshape(bs * _N, 1)
    edges = edges.astype(jnp.int32)
    edges_t = edges.T
    row_col = row_col.astype(jnp.int32)

    exw1i = ex_w1[:, :_EMBED].T + ex_b1[None, :]
    exw1rc = ex_w1[:, _EMBED:].T
    w6 = jnp.stack([ex_w2, ex_w3, mm_w2, mm_w3, li_w2, li_w3]
                   ).transpose(0, 2, 1)
    w2a = jnp.stack([mm_w1, li_w1]).transpose(0, 2, 1)
    wlstm = jnp.stack([lstm_wih, lstm_whh]).transpose(0, 2, 1)
    bias6 = jnp.stack([ex_b2, ex_b3, mm_b1, mm_b2, li_b2, li_b3])
    bg = (lstm_bih + lstm_bhh).reshape(1, 4 * _H)
    indeg = jnp.zeros((_N,), f32).at[edges[:, 1]].add(1.0)
    lb1 = li_b1[None, :] + indeg[:, None] * (mm_b3[None, :] @ li_w1[:, :_H].T)
    outw = jnp.zeros((_H, _EMBED), f32).at[:, :9].set(out_w.T)
    outb = jnp.full((1, _EMBED), _NEG, f32).at[0, :9].set(out_b)

    full = lambda shape: pl.BlockSpec(shape, lambda i: (0,) * len(shape))
    R = _BB * _N

    o_out, loss_out = pl.pallas_call(
        _rrn_kernel,
        grid=(bs // _BB,),
        in_specs=[
            pl.BlockSpec((R, 1), lambda i: (i, 0)),
            pl.BlockSpec((R, 1), lambda i: (i, 0)),
            full((e, 2)), full((2, e)), full((_N, 2)),
            full((_EMBED, _H)), full((32, _H)),
            full((6, _H, _H)), full((2, 2 * _H, _H)), full((2, _H, 4 * _H)),
            full((6, _H)), full((1, 4 * _H)), full((_N, _H)),
            full((_H, _EMBED)), full((1, _EMBED)),
        ],
        out_specs=[
            pl.BlockSpec((_BB, _N, _EMBED), lambda i: (i, 0, 0)),
            pl.BlockSpec((1, 1, 128), lambda i: (i, 0, 0)),
        ],
        out_shape=[
            jax.ShapeDtypeStruct((bs, _N, _EMBED), f32),
            jax.ShapeDtypeStruct((bs // _BB, 1, 128), f32),
        ],
        scratch_shapes=[
            pltpu.VMEM((_BB * _E, _H), f32),
            pltpu.VMEM((R, _H), f32),
        ],
        compiler_params=pltpu.CompilerParams(
            dimension_semantics=("parallel",)),
        interpret=_INTERPRET,
    )(inp2, y2, edges, edges_t, row_col,
      exw1i, exw1rc, w6, w2a, wlstm, bias6, bg, lb1, outw, outb)

    o = o_out.reshape(bs * _N, _EMBED)[:, :9]
    l = -jnp.sum(loss_out[:, 0, 0]) / (bs * _N)
    return (o, l)


# speedup vs baseline: 1.1866x; 1.1866x over previous
import jax
import jax.numpy as jnp
from jax.experimental import pallas as pl
from jax.experimental.pallas import tpu as pltpu

_EMBED = 16
_H = 96
_N = 64
_E = 1088
_STEPS = 4
_NEG = -1e9
_CDT = jnp.float32
_BB = 16
_INTERPRET = False


def _relu(v):
    return jnp.maximum(v, 0.0)


_DN = (((1,), (0,)), ((), ()))


def _mx(a, b):
    return jax.lax.dot_general(
        a.astype(_CDT), b.astype(_CDT), _DN,
        preferred_element_type=jnp.float32)


def _rrn_kernel(eo_ref, y1h_ref, erc_ref, gcat_ref, gdt_ref,
                exw1i_ref, exw1rc_ref, exw2_ref, exb2_ref,
                exw3_ref, exb3_ref,
                mm1a_ref, mm1b_ref, mmb1_ref, mm2_ref, mmb2_ref,
                mm3_ref,
                li1f_ref, li1x_ref, lib1_ref, li2_ref, lib2_ref,
                li3_ref, lib3_ref,
                wih_ref, whh_ref, bg_ref, outw_ref, outb_ref,
                o_ref, loss_ref, scr_ref, fin_ref):
    R = _BB * _N
    erc = erc_ref[...]
    gcat = gcat_ref[...]
    gdt = gdt_ref[...]

    exb2 = exb2_ref[...]
    exb3 = exb3_ref[...]
    mmb1 = mmb1_ref[...]
    mmb2 = mmb2_ref[...]
    lib2 = lib2_ref[...]
    lib3 = lib3_ref[...]
    bg = bg_ref[...]
    outb = outb_ref[...]

    exw1i = exw1i_ref[...]
    exw1rc = exw1rc_ref[...]
    exw2 = exw2_ref[...]
    exw3 = exw3_ref[...]
    li1x = li1x_ref[...]
    lib1 = lib1_ref[...]
    mm1a = mm1a_ref[...]
    mm1b = mm1b_ref[...]
    mm2w = mm2_ref[...]
    mm3w = mm3_ref[...]
    li1f = li1f_ref[...]
    li2w = li2_ref[...]
    li3w = li3_ref[...]
    wih = wih_ref[...]
    whh = whh_ref[...]
    outw = outw_ref[...]

    eo = eo_ref[...].reshape(R, _EMBED)
    y1h = y1h_ref[...].reshape(R, _EMBED)

    x = _relu(_mx(eo, exw1i) + _mx(erc, exw1rc))
    x = _relu(_mx(x, exw2) + exb2)
    x = _mx(x, exw3) + exb3
    xli = _mx(x, li1x) + lib1

    hm = x
    h = None
    c = None
    acc = jnp.zeros((), jnp.float32)
    B = range(_BB)
    for t in range(_STEPS):
        av = _mx(hm, mm1a) + mmb1
        bv = _mx(hm, mm1b)
        for j in B:
            abj = jnp.concatenate(
                [av[j * _N:(j + 1) * _N], bv[j * _N:(j + 1) * _N]], axis=0)
            scr_ref[pl.ds(j * _E, _E), :] = _mx(gcat, abj)
        m1 = _relu(scr_ref[...])
        m2 = _relu(_mx(m1, mm2w) + mmb2)
        msgs = _mx(m2, mm3w)
        for j in B:
            fin_ref[pl.ds(j * _N, _N), :] = _mx(
                gdt, msgs[j * _E:(j + 1) * _E])
        fin = fin_ref[...]
        li1 = _relu(_mx(fin, li1f) + xli)
        li2 = _relu(_mx(li1, li2w) + lib2)
        il = _mx(li2, li3w) + lib3
        gates = _mx(il, wih) + bg
        if t > 0:
            gates = gates + _mx(h, whh)
        ig = gates[:, 0:_H]
        fg = gates[:, _H:2 * _H]
        gg = gates[:, 2 * _H:3 * _H]
        og = gates[:, 3 * _H:4 * _H]
        newc = jax.nn.sigmoid(ig) * jnp.tanh(gg)
        if t > 0:
            newc = newc + jax.nn.sigmoid(fg) * c
        c = newc
        h = jax.nn.sigmoid(og) * jnp.tanh(newc)
        hm = h
        o16 = _mx(c, outw) + outb
        mmax = jnp.max(o16, axis=1, keepdims=True)
        lse = jnp.log(jnp.sum(jnp.exp(o16 - mmax), axis=1,
                              keepdims=True)) + mmax
        acc = acc + jnp.sum((o16 - lse) * y1h)

    o_ref[...] = o16.reshape(_BB, _N, _EMBED)
    loss_ref[...] = jnp.broadcast_to(acc, (1, 1, 128)).astype(jnp.float32)


def kernel(inp, y_true, edges, row_col,
           ex_w1, ex_b1, ex_w2, ex_b2, ex_w3, ex_b3,
           mm_w1, mm_b1, mm_w2, mm_b2, mm_w3, mm_b3,
           li_w1, li_b1, li_w2, li_b2, li_w3, li_b3,
           out_w, out_b, lstm_wih, lstm_whh, lstm_bih, lstm_bhh):
    f32 = jnp.float32
    bs = inp.shape[0]
    e = edges.shape[0]
    inp = inp.astype(jnp.int32)
    y2 = y_true.astype(jnp.int32).reshape(bs, _N)

    eo = jax.nn.one_hot(inp, _EMBED, dtype=f32)
    y1h = jax.nn.one_hot(y2, _EMBED, dtype=f32)
    erc = jnp.tile(jnp.concatenate(
        [jax.nn.one_hot(row_col[:, 0], _EMBED, dtype=f32),
         jax.nn.one_hot(row_col[:, 1], _EMBED, dtype=f32)], axis=1),
        (_BB, 1))
    gsrc = jax.nn.one_hot(edges[:, 0], _N, dtype=f32)
    gdst = jax.nn.one_hot(edges[:, 1], _N, dtype=f32)
    gcat = jnp.concatenate([gsrc, gdst], axis=1).astype(_CDT)
    gdt = gdst.T.astype(_CDT)

    cd = _CDT
    exw1i = (ex_w1[:, :_EMBED].T + ex_b1[None, :]).astype(cd)
    exw1rc = ex_w1[:, _EMBED:].T.astype(cd)
    exw2 = ex_w2.T.astype(cd)
    exw3 = ex_w3.T.astype(cd)
    mm1a = mm_w1[:, :_H].T.astype(cd)
    mm1b = mm_w1[:, _H:].T.astype(cd)
    mm2 = mm_w2.T.astype(cd)
    mm3 = mm_w3.T.astype(cd)
    li1f = li_w1[:, :_H].T.astype(cd)
    li1x = li_w1[:, _H:].T.astype(cd)
    li2 = li_w2.T.astype(cd)
    li3 = li_w3.T.astype(cd)
    wih = lstm_wih.T.astype(cd)
    whh = lstm_whh.T.astype(cd)
    bg = (lstm_bih + lstm_bhh).reshape(1, 4 * _H)
    outw = jnp.zeros((_H, _EMBED), f32).at[:, :9].set(out_w.T).astype(cd)
    outb = jnp.full((1, _EMBED), _NEG, f32).at[0, :9].set(out_b)

    b2 = ex_b2.reshape(1, _H)
    b3 = ex_b3.reshape(1, _H)
    mb1 = mm_b1.reshape(1, _H)
    mb2 = mm_b2.reshape(1, _H)
    indeg = jnp.sum(gdst, axis=0)
    lb1 = jnp.tile(
        li_b1[None, :] + indeg[:, None] * (mm_b3[None, :] @ li_w1[:, :_H].T),
        (_BB, 1))
    lb2 = li_b2.reshape(1, _H)
    lb3 = li_b3.reshape(1, _H)

    full = lambda shape: pl.BlockSpec(shape, lambda i: (0,) * len(shape))
    per_b = pl.BlockSpec((_BB, _N, _EMBED), lambda i: (i, 0, 0))
    R = _BB * _N

    o_out, loss_out = pl.pallas_call(
        _rrn_kernel,
        grid=(bs // _BB,),
        in_specs=[
            per_b, per_b, full((R, 32)), full((e, 128)), full((_N, e)),
            full((_EMBED, _H)), full((32, _H)),
            full((_H, _H)), full((1, _H)), full((_H, _H)), full((1, _H)),
            full((_H, _H)), full((_H, _H)), full((1, _H)),
            full((_H, _H)), full((1, _H)), full((_H, _H)),
            full((_H, _H)), full((_H, _H)), full((R, _H)),
            full((_H, _H)), full((1, _H)), full((_H, _H)), full((1, _H)),
            full((_H, 4 * _H)), full((_H, 4 * _H)), full((1, 4 * _H)),
            full((_H, _EMBED)), full((1, _EMBED)),
        ],
        out_specs=[
            pl.BlockSpec((_BB, _N, _EMBED), lambda i: (i, 0, 0)),
            pl.BlockSpec((1, 1, 128), lambda i: (i, 0, 0)),
        ],
        out_shape=[
            jax.ShapeDtypeStruct((bs, _N, _EMBED), f32),
            jax.ShapeDtypeStruct((bs // _BB, 1, 128), f32),
        ],
        scratch_shapes=[
            pltpu.VMEM((_BB * _E, _H), f32),
            pltpu.VMEM((R, _H), f32),
        ],
        compiler_params=pltpu.CompilerParams(
            dimension_semantics=("parallel",)),
        interpret=_INTERPRET,
    )(eo, y1h, erc, gcat, gdt,
      exw1i, exw1rc, exw2, b2, exw3, b3,
      mm1a, mm1b, mb1, mm2, mb2, mm3,
      li1f, li1x, lb1, li2, lb2, li3, lb3,
      wih, whh, bg, outw, outb)

    o = o_out.reshape(bs * _N, _EMBED)[:, :9]
    l = -jnp.sum(loss_out[:, 0, 0]) / (bs * _N)
    return (o, l)
